# Initial kernel scaffold; baseline (speedup 1.0000x reference)
import jax
import jax.numpy as jnp
from jax.experimental import pallas as pl


def kernel(query, key, value):
    raise NotImplementedError("write your pallas kernel here")


# speedup vs baseline: 3.5617x; 3.5617x over previous
import functools
import math

import jax
import jax.numpy as jnp
from jax.experimental import pallas as pl

_NUM_LANDMARKS = 64
_WINDOW = 64
_HALF = _WINDOW // 2


def _ssa_block_kernel(q_ref, kpad_ref, vpad_ref, o_ref, *, bq, seq_len, stride):
    i = pl.program_id(1)
    d = q_ref.shape[-1]
    scale = 1.0 / math.sqrt(d)

    q = q_ref[0]

    kwin = kpad_ref[0, pl.ds(i * bq, bq + _HALF), :]
    vwin = vpad_ref[0, pl.ds(i * bq, bq + _HALF), :]

    scores = jax.lax.dot_general(
        q, kwin, (((1,), (1,)), ((), ())), preferred_element_type=jnp.float32
    ) * scale

    r = jax.lax.broadcasted_iota(jnp.int32, (bq, bq + _HALF), 0)
    c = jax.lax.broadcasted_iota(jnp.int32, (bq, bq + _HALF), 1)
    band = (c >= r) & (c <= r + _HALF)
    is_pad = (i * bq + c) < _HALF
    scores = jnp.where(band, jnp.where(is_pad, 0.0, scores), -jnp.inf)

    m = jnp.max(scores, axis=1, keepdims=True)
    e = jnp.exp(scores - m)
    denom = jnp.sum(e, axis=1, keepdims=True)
    local = jax.lax.dot_general(
        e, vwin, (((1,), (0,)), ((), ())), preferred_element_type=jnp.float32
    ) / denom

    lm_k = kpad_ref[0, _HALF:, :].reshape(_NUM_LANDMARKS, stride, d)[:, 0, :]
    lm_v = vpad_ref[0, _HALF:, :].reshape(_NUM_LANDMARKS, stride, d)[:, 0, :]

    lm_scores = jax.lax.dot_general(
        q, lm_k, (((1,), (1,)), ((), ())), preferred_element_type=jnp.float32
    ) * scale
    s_idx = i * bq + jax.lax.broadcasted_iota(jnp.int32, (bq, _NUM_LANDMARKS), 0)
    l_pos = jax.lax.broadcasted_iota(jnp.int32, (bq, _NUM_LANDMARKS), 1) * stride
    lm_scores = jnp.where(l_pos > s_idx, -jnp.inf, lm_scores)

    m2 = jnp.max(lm_scores, axis=1, keepdims=True)
    e2 = jnp.exp(lm_scores - m2)
    denom2 = jnp.sum(e2, axis=1, keepdims=True)
    glob = jax.lax.dot_general(
        e2, lm_v, (((1,), (0,)), ((), ())), preferred_element_type=jnp.float32
    ) / denom2

    o_ref[0] = (local + glob).astype(o_ref.dtype)


@jax.jit
def kernel(query, key, value):
    b, h, s, d = query.shape
    assert b == 1
    bq = 256
    stride = s // _NUM_LANDMARKS

    q = query[0]
    kpad = jnp.pad(key[0], ((0, 0), (_HALF, 0), (0, 0)))
    vpad = jnp.pad(value[0], ((0, 0), (_HALF, 0), (0, 0)))

    grid = (h, s // bq)
    out = pl.pallas_call(
        functools.partial(_ssa_block_kernel, bq=bq, seq_len=s, stride=stride),
        grid=grid,
        in_specs=[
            pl.BlockSpec((1, bq, d), lambda hh, ii: (hh, ii, 0)),
            pl.BlockSpec((1, s + _HALF, d), lambda hh, ii: (hh, 0, 0)),
            pl.BlockSpec((1, s + _HALF, d), lambda hh, ii: (hh, 0, 0)),
        ],
        out_specs=pl.BlockSpec((1, bq, d), lambda hh, ii: (hh, ii, 0)),
        out_shape=jax.ShapeDtypeStruct((h, s, d), query.dtype),
    )(q, kpad, vpad)
    return out[None]


# speedup vs baseline: 3.6815x; 1.0336x over previous
import functools
import math

import jax
import jax.numpy as jnp
from jax.experimental import pallas as pl
from jax.experimental.pallas import tpu as pltpu

_NUM_LANDMARKS = 64
_WINDOW = 64
_HALF = _WINDOW // 2


def _ssa_block_kernel(q_ref, k_ref, v_ref, o_ref, lmk_ref, lmv_ref, *, bq, stride):
    i = pl.program_id(1)
    d = q_ref.shape[-1]
    scale = 1.0 / math.sqrt(d)
    kw = bq + _HALF

    q = q_ref[0]

    @pl.when(i == 0)
    def _():
        lmk_ref[...] = k_ref[0].reshape(_NUM_LANDMARKS, stride, d)[:, 0, :]
        lmv_ref[...] = v_ref[0].reshape(_NUM_LANDMARKS, stride, d)[:, 0, :]

    start = jnp.maximum(i * bq - _HALF, 0)
    kwin = k_ref[0, pl.ds(start, kw), :]
    vwin = v_ref[0, pl.ds(start, kw), :]

    scores = jax.lax.dot_general(
        q, kwin, (((1,), (1,)), ((), ())), preferred_element_type=jnp.float32
    ) * scale

    r = jax.lax.broadcasted_iota(jnp.int32, (bq, kw), 0)
    c = jax.lax.broadcasted_iota(jnp.int32, (bq, kw), 1)
    s_abs = i * bq + r
    p_abs = start + c
    band = (p_abs >= s_abs - _HALF) & (p_abs <= s_abs)
    scores = jnp.where(band, scores, -jnp.inf)

    s_col = i * bq + jax.lax.broadcasted_iota(jnp.int32, (bq, 1), 0)
    n_pad = jnp.maximum(_HALF - s_col, 0).astype(jnp.float32)

    m = jnp.max(scores, axis=1, keepdims=True)
    m = jnp.where(n_pad > 0, jnp.maximum(m, 0.0), m)
    e = jnp.exp(scores - m)
    denom = jnp.sum(e, axis=1, keepdims=True) + n_pad * jnp.exp(-m)
    local = jax.lax.dot_general(
        e, vwin, (((1,), (0,)), ((), ())), preferred_element_type=jnp.float32
    ) / denom

    lm_scores = jax.lax.dot_general(
        q, lmk_ref[...], (((1,), (1,)), ((), ())),
        preferred_element_type=jnp.float32,
    ) * scale
    l_pos = jax.lax.broadcasted_iota(jnp.int32, (bq, _NUM_LANDMARKS), 1) * stride
    lm_scores = jnp.where(l_pos > s_col, -jnp.inf, lm_scores)

    m2 = jnp.max(lm_scores, axis=1, keepdims=True)
    e2 = jnp.exp(lm_scores - m2)
    denom2 = jnp.sum(e2, axis=1, keepdims=True)
    glob = jax.lax.dot_general(
        e2, lmv_ref[...], (((1,), (0,)), ((), ())),
        preferred_element_type=jnp.float32,
    ) / denom2

    o_ref[0] = (local + glob).astype(o_ref.dtype)


@jax.jit
def kernel(query, key, value):
    b, h, s, d = query.shape
    assert b == 1
    bq = 256
    stride = s // _NUM_LANDMARKS

    grid = (h, s // bq)
    out = pl.pallas_call(
        functools.partial(_ssa_block_kernel, bq=bq, stride=stride),
        grid=grid,
        in_specs=[
            pl.BlockSpec((1, bq, d), lambda hh, ii: (hh, ii, 0)),
            pl.BlockSpec((1, s, d), lambda hh, ii: (hh, 0, 0)),
            pl.BlockSpec((1, s, d), lambda hh, ii: (hh, 0, 0)),
        ],
        out_specs=pl.BlockSpec((1, bq, d), lambda hh, ii: (hh, ii, 0)),
        out_shape=jax.ShapeDtypeStruct((h, s, d), query.dtype),
        scratch_shapes=[
            pltpu.VMEM((_NUM_LANDMARKS, d), jnp.float32),
            pltpu.VMEM((_NUM_LANDMARKS, d), jnp.float32),
        ],
    )(query[0], key[0], value[0])
    return out[None]
